# baseline (speedup 1.0000x reference)
import jax
import jax.numpy as jnp
from jax import lax
from jax.experimental import pallas as pl
from jax.experimental.pallas import tpu as pltpu
from jax.experimental.pallas import tpu_sc as plsc

VOCAB = 1000000
EMBED_DIM = 64
BATCH = 16384

NUM_CORES = 2
NUM_SUBCORES = 16
LANES = 16
NUM_WORKERS = NUM_CORES * NUM_SUBCORES
B_PER_W = BATCH // NUM_WORKERS
WIDE = 2 * EMBED_DIM

OWN_W = 31232
STRIPE_W = 256
N_FULL = OWN_W // STRIPE_W
LAST_EXTRA = 2
TAIL_W = VOCAB - (NUM_WORKERS - 1) * OWN_W - (N_FULL + LAST_EXTRA) * STRIPE_W
N_T = N_FULL + LAST_EXTRA + 1


def _scan_body(tbl_hbm, tail_hbm, idx_hbm, stage_hbm,
               buf, idx_v, lvals, lids, rowstage, gidx, hloc, sem, esem):
    wid = lax.axis_index("s") * NUM_CORES + lax.axis_index("c")
    mylo = wid * OWN_W
    is_last = wid == NUM_WORKERS - 1
    myhi = lax.select(is_last, jnp.int32(VOCAB), mylo + OWN_W)
    nfull = lax.select(is_last, jnp.int32(N_FULL + LAST_EXTRA),
                       jnp.int32(N_FULL))

    pltpu.sync_copy(idx_hbm, idx_v)

    lane = lax.iota(jnp.int32, LANES)

    def filt(g, cnt):
        vec = idx_v[pl.ds(g * LANES, LANES)]
        m = jnp.logical_and(vec >= mylo, vec < myhi)
        plsc.store_compressed(lvals.at[pl.ds(cnt, LANES)], vec - mylo, mask=m)
        plsc.store_compressed(lids.at[pl.ds(cnt, LANES)],
                              g * LANES + lane, mask=m)
        return cnt + plsc.all_reduce_population_count(m)[0]

    cnt = lax.fori_loop(0, BATCH // LANES, filt, jnp.int32(0))
    ngroups = lax.div(cnt + (LANES - 1), LANES)

    def fire(t):
        par = lax.rem(t, 2)

        @pl.when(t < nfull)
        def _():
            colw = pl.multiple_of(mylo + t * STRIPE_W, 128)
            for a in range(EMBED_DIM // 8):
                pltpu.async_copy(
                    tbl_hbm.at[pl.ds(a * 8, 8), pl.ds(colw, STRIPE_W)],
                    buf.at[par, pl.ds(a * 8, 8), :], sem.at[par])

        @pl.when(jnp.logical_and(is_last, t == nfull))
        def _():
            pltpu.async_copy(tail_hbm, buf.at[par, :, pl.ds(0, 128)],
                             sem.at[par])

    def drain(t):
        par = lax.rem(t, 2)

        @pl.when(t < nfull)
        def _():
            for a in range(EMBED_DIM // 8):
                pltpu.make_async_copy(
                    tbl_hbm.at[pl.ds(a * 8, 8), pl.ds(0, STRIPE_W)],
                    buf.at[par, pl.ds(a * 8, 8), :], sem.at[par]).wait()

        @pl.when(jnp.logical_and(is_last, t == nfull))
        def _():
            pltpu.make_async_copy(tail_hbm, buf.at[par, :, pl.ds(0, 128)],
                                  sem.at[par]).wait()

    def do_stripe(t, par):
        lo = t * STRIPE_W
        parv = jnp.full((LANES,), 0, jnp.int32) + par

        def group_body(g, carry2):
            gsl = pl.ds(g * LANES, LANES)
            vec = lvals[gsl]
            pos = g * LANES + lane
            m = jnp.logical_and(
                jnp.logical_and(vec >= lo, vec < lo + STRIPE_W), pos < cnt)
            npop = plsc.all_reduce_population_count(m)[0]

            @pl.when(npop > 0)
            def _():
                plsc.store_compressed(hloc.at[pl.ds(0, LANES)], vec - lo, mask=m)
                plsc.store_compressed(gidx.at[pl.ds(0, LANES)], lids[gsl], mask=m)
                c_loc = lax.max(jnp.zeros((LANES,), jnp.int32),
                                lax.min(hloc[...],
                                        jnp.full((LANES,), STRIPE_W - 1,
                                                 jnp.int32)))
                hids = lax.select(lane < npop, gidx[...],
                                  jnp.full((LANES,), -1, jnp.int32))
                gidx[...] = hids
                for j in range(EMBED_DIM):
                    jv = jnp.full((LANES,), j, jnp.int32)
                    vv = plsc.load_gather(buf, [parv, jv, c_loc])
                    plsc.store_scatter(rowstage, [lane, jv], vv)
                pltpu.async_copy(
                    rowstage,
                    stage_hbm.at[plsc.Indices(gidx, ignored_value=-1)],
                    esem).wait()

            return carry2

        lax.fori_loop(0, ngroups, group_body, 0)

    fire(0)

    def stripe_loop(t, carry):
        @pl.when(t + 1 < N_T)
        def _():
            fire(t + 1)

        drain(t)
        par = lax.rem(t, 2)

        @pl.when(jnp.logical_or(t < nfull,
                                jnp.logical_and(is_last, t == nfull)))
        def _():
            do_stripe(t, par)

        return carry

    lax.fori_loop(0, N_T, stripe_loop, 0)


DCHUNK = 128


def _dot_body(vstage_hbm, ustage_hbm, out_hbm, vrows, urows, res_v, sem):
    wid = lax.axis_index("s") * NUM_CORES + lax.axis_index("c")
    base = wid * B_PER_W

    lane = lax.iota(jnp.int32, LANES)

    def chunk_body(k, carry):
        rbase = base + k * DCHUNK
        cp1 = pltpu.async_copy(vstage_hbm.at[pl.ds(rbase, DCHUNK)], vrows,
                               sem)
        cp2 = pltpu.async_copy(ustage_hbm.at[pl.ds(rbase, DCHUNK)], urows,
                               sem)
        cp1.wait()
        cp2.wait()
        for g in range(DCHUNK // LANES):
            slot = g * LANES + lane
            acc = jnp.zeros((LANES,), jnp.float32)
            for j in range(EMBED_DIM):
                jv = jnp.full((LANES,), j, jnp.int32)
                vv = plsc.load_gather(vrows, [slot, jv])
                uu = plsc.load_gather(urows, [slot, jv])
                acc = acc + vv * uu
            res_v[pl.ds(k * DCHUNK + g * LANES, LANES)] = acc
        return carry

    lax.fori_loop(0, B_PER_W // DCHUNK, chunk_body, 0)

    pltpu.sync_copy(res_v, out_hbm.at[pl.ds(base, B_PER_W)])


def _make_mesh():
    return plsc.VectorSubcoreMesh(
        core_axis_name="c", subcore_axis_name="s",
        num_cores=NUM_CORES, num_subcores=NUM_SUBCORES)


def _scan_call(tbl, tail, idx):
    k = pl.kernel(
        _scan_body,
        out_type=jax.ShapeDtypeStruct((BATCH, WIDE), jnp.float32),
        mesh=_make_mesh(),
        compiler_params=pltpu.CompilerParams(needs_layout_passes=False),
        scratch_types=[
            pltpu.VMEM((2, EMBED_DIM, STRIPE_W), jnp.float32),
            pltpu.VMEM((BATCH,), jnp.int32),
            pltpu.VMEM((BATCH + LANES,), jnp.int32),
            pltpu.VMEM((BATCH + LANES,), jnp.int32),
            pltpu.VMEM((LANES, WIDE), jnp.float32),
            pltpu.VMEM((LANES,), jnp.int32),
            pltpu.VMEM((LANES,), jnp.int32),
            pltpu.SemaphoreType.DMA((2,)),
            pltpu.SemaphoreType.DMA,
        ],
    )
    return k(tbl, tail, idx)


def _dot_call(vstage, ustage):
    k = pl.kernel(
        _dot_body,
        out_type=jax.ShapeDtypeStruct((BATCH,), jnp.float32),
        mesh=_make_mesh(),
        compiler_params=pltpu.CompilerParams(needs_layout_passes=False),
        scratch_types=[
            pltpu.VMEM((DCHUNK, WIDE), jnp.float32),
            pltpu.VMEM((DCHUNK, WIDE), jnp.float32),
            pltpu.VMEM((B_PER_W,), jnp.float32),
            pltpu.SemaphoreType.DMA,
        ],
    )
    return k(vstage, ustage)


def _tail_view(tbl):
    t = tbl[VOCAB - 64:].T
    return jnp.pad(t, ((0, 0), (0, 64)))


@jax.jit
def kernel(center_ids, context_ids, in_embed, out_embed):
    cidx = center_ids.astype(jnp.int32)
    xidx = context_ids.astype(jnp.int32)
    vstage = _scan_call(in_embed.T, _tail_view(in_embed), cidx)
    ustage = _scan_call(out_embed.T, _tail_view(out_embed), xidx)
    return _dot_call(vstage, ustage)
